# baseline (speedup 1.0000x reference)
import jax
import jax.numpy as jnp
from jax.experimental import pallas as pl


_BN = 256


def _add_kernel(enc_ref, pos_ref, out_ref):
    out_ref[...] = enc_ref[...] + pos_ref[...]


def kernel(encoded_tokens, pos_table):
    b, n, d = encoded_tokens.shape
    num_n = n // _BN
    return pl.pallas_call(
        _add_kernel,
        grid=(num_n,),
        in_specs=[
            pl.BlockSpec((b, _BN, d), lambda i: (0, i, 0)),
            pl.BlockSpec((1, _BN, d), lambda i: (0, i, 0)),
        ],
        out_specs=pl.BlockSpec((b, _BN, d), lambda i: (0, i, 0)),
        out_shape=jax.ShapeDtypeStruct((b, n, d), encoded_tokens.dtype),
    )(encoded_tokens, pos_table[None])


# speedup vs baseline: 1.0202x; 1.0202x over previous
import jax
import jax.numpy as jnp
from jax.experimental import pallas as pl


_BN = 1024


def _add_kernel(enc_ref, pos_ref, out_ref):
    out_ref[...] = enc_ref[...] + pos_ref[...]


def kernel(encoded_tokens, pos_table):
    b, n, d = encoded_tokens.shape
    num_n = n // _BN
    return pl.pallas_call(
        _add_kernel,
        grid=(num_n,),
        in_specs=[
            pl.BlockSpec((b, _BN, d), lambda i: (0, i, 0)),
            pl.BlockSpec((1, _BN, d), lambda i: (0, i, 0)),
        ],
        out_specs=pl.BlockSpec((b, _BN, d), lambda i: (0, i, 0)),
        out_shape=jax.ShapeDtypeStruct((b, n, d), encoded_tokens.dtype),
    )(encoded_tokens, pos_table[None])


# speedup vs baseline: 1.0274x; 1.0070x over previous
import jax
import jax.numpy as jnp
from jax.experimental import pallas as pl


_BN = 512


def _add_kernel(enc_ref, pos_ref, out_ref):
    out_ref[...] = enc_ref[...] + pos_ref[...]


def kernel(encoded_tokens, pos_table):
    b, n, d = encoded_tokens.shape
    num_n = n // _BN
    return pl.pallas_call(
        _add_kernel,
        grid=(num_n,),
        in_specs=[
            pl.BlockSpec((b, _BN, d), lambda i: (0, i, 0)),
            pl.BlockSpec((1, _BN, d), lambda i: (0, i, 0)),
        ],
        out_specs=pl.BlockSpec((b, _BN, d), lambda i: (0, i, 0)),
        out_shape=jax.ShapeDtypeStruct((b, n, d), encoded_tokens.dtype),
    )(encoded_tokens, pos_table[None])
